# baseline (speedup 1.0000x reference)
import functools

import jax
import jax.numpy as jnp
import numpy as np
from jax import lax
from jax.experimental import pallas as pl
from jax.experimental.pallas import tpu as pltpu
from jax.experimental.pallas import tpu_sc as plsc

BS = 2
NC = 6
NPB = 50000
C_IMG = 256
D_OUT = 128
H_F = 16
W_F = 44
DS = 16
N_PTS = BS * NPB
N_PAD = 102400
PIX = H_F * W_F
ZERO_ROW = BS * NC * PIX
TBL_ROWS = ZERO_ROW + 8
NWORK = 32
CHUNK = 128
CHUNKS_PER_W = N_PAD // (NWORK * CHUNK)
VOXEL_SIZE = np.array([0.1, 0.1, 0.2], dtype=np.float32)
PC_RANGE = np.array([-51.2, -51.2, -5.0], dtype=np.float32)
D_MIN, D_MAX = 1.0, 60.0


def _table_body(f_ref, w_ref, o_ref):
    o_ref[0] = lax.dot_general(
        f_ref[0], w_ref[...],
        dimension_numbers=(((0,), (0,)), ((), ())),
        preferred_element_type=jnp.float32,
    )


def _build_tables(feats2d, w_t):
    t12 = pl.pallas_call(
        _table_body,
        grid=(BS * NC,),
        in_specs=[
            pl.BlockSpec((1, C_IMG, PIX), lambda g: (g, 0, 0)),
            pl.BlockSpec((C_IMG, D_OUT), lambda g: (0, 0)),
        ],
        out_specs=pl.BlockSpec((1, PIX, D_OUT), lambda g: (g, 0, 0)),
        out_shape=jax.ShapeDtypeStruct((BS * NC, PIX, D_OUT), jnp.float32),
    )(feats2d, w_t)
    return jnp.concatenate(
        [t12.reshape(BS * NC * PIX, D_OUT),
         jnp.zeros((TBL_ROWS - ZERO_ROW, D_OUT), jnp.float32)], axis=0)


def _bf(x):
    return x.astype(jnp.bfloat16).astype(jnp.float32)


def _proj_body(x_ref, y_ref, z_ref, bt_ref, ir_ref, ab_ref, tt_ref, pr_ref,
               pt_ref, out_ref):
    for b in range(BS):
        sl = pl.ds(b * NPB, NPB)
        x1 = (x_ref[sl] * float(VOXEL_SIZE[0]) + float(PC_RANGE[0])) - bt_ref[b, 0]
        y1 = (y_ref[sl] * float(VOXEL_SIZE[1]) + float(PC_RANGE[1])) - bt_ref[b, 1]
        z1 = (z_ref[sl] * float(VOXEL_SIZE[2]) + float(PC_RANGE[2])) - bt_ref[b, 2]
        xb, yb, zb = _bf(x1), _bf(y1), _bf(z1)
        s0 = xb * ir_ref[b, 0, 0] + yb * ir_ref[b, 0, 1] + zb * ir_ref[b, 0, 2]
        s1 = xb * ir_ref[b, 1, 0] + yb * ir_ref[b, 1, 1] + zb * ir_ref[b, 1, 2]
        s2 = xb * ir_ref[b, 2, 0] + yb * ir_ref[b, 2, 1] + zb * ir_ref[b, 2, 2]
        sb0, sb1, sb2 = _bf(s0), _bf(s1), _bf(s2)
        for c in range(NC):
            p0 = tt_ref[b, c, 0] + sb0 * ab_ref[b, c, 0, 0] + sb1 * ab_ref[b, c, 0, 1] + sb2 * ab_ref[b, c, 0, 2]
            p1 = tt_ref[b, c, 1] + sb0 * ab_ref[b, c, 1, 0] + sb1 * ab_ref[b, c, 1, 1] + sb2 * ab_ref[b, c, 1, 2]
            p2 = tt_ref[b, c, 2] + sb0 * ab_ref[b, c, 2, 0] + sb1 * ab_ref[b, c, 2, 1] + sb2 * ab_ref[b, c, 2, 2]
            u = p0 / p2
            v = p1 / p2
            ub, vb, db = _bf(u), _bf(v), _bf(p2)
            q0 = pt_ref[b, c, 0] + ub * pr_ref[b, c, 0, 0] + vb * pr_ref[b, c, 0, 1] + db * pr_ref[b, c, 0, 2]
            q1 = pt_ref[b, c, 1] + ub * pr_ref[b, c, 1, 0] + vb * pr_ref[b, c, 1, 1] + db * pr_ref[b, c, 1, 2]
            q2 = pt_ref[b, c, 2] + ub * pr_ref[b, c, 2, 0] + vb * pr_ref[b, c, 2, 1] + db * pr_ref[b, c, 2, 2]
            cx = jnp.round(q0 / float(DS))
            cy = jnp.round(q1 / float(DS))
            kept = ((cx >= 0) & (cx < W_F) & (cy >= 0) & (cy < H_F)
                    & (q2 < D_MAX) & (q2 >= D_MIN))
            cxi = jnp.clip(jnp.where(jnp.isnan(cx), 0.0, cx), 0.0, W_F - 1.0).astype(jnp.int32)
            cyi = jnp.clip(jnp.where(jnp.isnan(cy), 0.0, cy), 0.0, H_F - 1.0).astype(jnp.int32)
            g = (b * NC + c) * PIX + cyi * W_F + cxi
            out_ref[c, sl] = jnp.where(kept, g, ZERO_ROW)


def _project_indices(xs, ys, zs, bt, ir, ab, tt, pr, pt):
    return pl.pallas_call(
        _proj_body,
        in_specs=[pl.BlockSpec(memory_space=pltpu.VMEM)] * 3
        + [pl.BlockSpec(memory_space=pltpu.SMEM)] * 6,
        out_specs=pl.BlockSpec(memory_space=pltpu.VMEM),
        out_shape=jax.ShapeDtypeStruct((NC, N_PTS), jnp.int32),
    )(xs, ys, zs, bt, ir, ab, tt, pr, pt)


@functools.cache
def _make_sc_gather_acc():
    return functools.partial(
        pl.kernel,
        out_type=jax.ShapeDtypeStruct((N_PAD, D_OUT), jnp.float32),
        mesh=plsc.VectorSubcoreMesh(core_axis_name="c", subcore_axis_name="s"),
        scratch_types=[
            pltpu.VMEM((CHUNKS_PER_W, NC * CHUNK), jnp.int32),
            pltpu.VMEM((NC * CHUNK, D_OUT), jnp.float32),
            pltpu.SemaphoreType.DMA,
        ],
        compiler_params=pltpu.CompilerParams(use_tc_tiling_on_sc=False),
    )(_sc_body)


def _sc_body(table_hbm, gidx_hbm, out_hbm, idx_v, buf_v, sem):
    wid = lax.axis_index("s") * 2 + lax.axis_index("c")
    pltpu.sync_copy(gidx_hbm.at[wid], idx_v)

    def chunk_body(i, carry):
        pltpu.async_copy(table_hbm.at[idx_v.at[i]], buf_v, sem).wait()

        def point_body(p, carry2):
            for d in range(D_OUT // 16):
                dsl = pl.ds(d * 16, 16)
                s = buf_v[p, dsl]
                for c in range(1, NC):
                    s = s + buf_v[c * CHUNK + p, dsl]
                buf_v[p, dsl] = s
            return carry2

        lax.fori_loop(0, CHUNK, point_body, 0)
        pltpu.sync_copy(
            buf_v.at[pl.ds(0, CHUNK)],
            out_hbm.at[pl.ds(wid * (CHUNKS_PER_W * CHUNK) + i * CHUNK, CHUNK)])
        return carry

    lax.fori_loop(0, CHUNKS_PER_W, chunk_body, 0)


def kernel(voxel_features, voxel_coords, img_feats, rots, trans, intrins,
           post_rots, post_trans, bda, lidar2cam, W, imgs):
    f32 = jnp.float32
    bf16 = jnp.bfloat16
    eye4 = jnp.eye(4, dtype=f32)
    c2i = jnp.tile(eye4, (BS, NC, 1, 1))
    c2i = c2i.at[:, :, :3, :3].set(intrins)
    l2i = jnp.einsum("bcij,bckj->bcik", c2i, lidar2cam)
    ab = l2i[:, :, :3, :3].astype(bf16).astype(f32)
    tt = l2i[:, :, :3, 3]
    ir = jnp.linalg.inv(bda[:, :3, :3]).astype(bf16).astype(f32)
    bt = bda[:, :3, 3]
    prb = post_rots.astype(f32).astype(bf16).astype(f32)
    ptf = post_trans.astype(f32)

    xs = voxel_coords[:, 3].astype(f32)
    ys = voxel_coords[:, 2].astype(f32)
    zs = voxel_coords[:, 1].astype(f32)

    feats2d = img_feats.reshape(BS * NC, C_IMG, PIX).astype(bf16).astype(f32)
    table = _build_tables(feats2d, W.T.astype(bf16).astype(f32))

    gidx = _project_indices(xs, ys, zs, bt, ir, ab, tt, prb, ptf)
    gidx_pad = jnp.pad(gidx, ((0, 0), (0, N_PAD - N_PTS)),
                       constant_values=ZERO_ROW)
    gidx3 = (gidx_pad.reshape(NC, NWORK, CHUNKS_PER_W, CHUNK)
             .transpose(1, 2, 0, 3)
             .reshape(NWORK, CHUNKS_PER_W, NC * CHUNK))

    img_pad = _make_sc_gather_acc()(table, gidx3)
    img_voxel = img_pad[:N_PTS]

    out_features = jnp.concatenate([voxel_features, img_voxel], axis=0)
    out_coords = jnp.concatenate([voxel_coords, voxel_coords], axis=0)
    return (out_features, out_coords)


# speedup vs baseline: 41.8750x; 41.8750x over previous
import functools

import jax
import jax.numpy as jnp
import numpy as np
from jax import lax
from jax.experimental import pallas as pl
from jax.experimental.pallas import tpu as pltpu
from jax.experimental.pallas import tpu_sc as plsc

BS = 2
NC = 6
NPB = 50000
C_IMG = 256
D_OUT = 128
H_F = 16
W_F = 44
DS = 16
N_PTS = BS * NPB
PIX = H_F * W_F
LOC_ZERO = NC * PIX
LOC_ROWS = LOC_ZERO + 8
DSLICE = 16
NDG = D_OUT // DSLICE
NPG = 4
PG_PTS = 25600
N_PAD = NPG * PG_PTS
CH = 1280
NCHUNK = PG_PTS // CH
GROUPS = CH // 16
VOXEL_SIZE = np.array([0.1, 0.1, 0.2], dtype=np.float32)
PC_RANGE = np.array([-51.2, -51.2, -5.0], dtype=np.float32)
D_MIN, D_MAX = 1.0, 60.0


def _table_body(f_ref, w_ref, o_ref):
    o_ref[0] = lax.dot_general(
        f_ref[0], w_ref[...],
        dimension_numbers=(((0,), (0,)), ((), ())),
        preferred_element_type=jnp.float32,
    )


def _build_tables(feats2d, w_t):
    return pl.pallas_call(
        _table_body,
        grid=(BS * NC,),
        in_specs=[
            pl.BlockSpec((1, C_IMG, PIX), lambda g: (g, 0, 0)),
            pl.BlockSpec((C_IMG, D_OUT), lambda g: (0, 0)),
        ],
        out_specs=pl.BlockSpec((1, PIX, D_OUT), lambda g: (g, 0, 0)),
        out_shape=jax.ShapeDtypeStruct((BS * NC, PIX, D_OUT), jnp.float32),
    )(feats2d, w_t)


def _bf(x):
    return x.astype(jnp.bfloat16).astype(jnp.float32)


def _proj_body(x_ref, y_ref, z_ref, bt_ref, ir_ref, ab_ref, tt_ref, pr_ref,
               pt_ref, out_ref):
    for b in range(BS):
        sl = pl.ds(b * NPB, NPB)
        x1 = (x_ref[sl] * float(VOXEL_SIZE[0]) + float(PC_RANGE[0])) - bt_ref[b, 0]
        y1 = (y_ref[sl] * float(VOXEL_SIZE[1]) + float(PC_RANGE[1])) - bt_ref[b, 1]
        z1 = (z_ref[sl] * float(VOXEL_SIZE[2]) + float(PC_RANGE[2])) - bt_ref[b, 2]
        xb, yb, zb = _bf(x1), _bf(y1), _bf(z1)
        s0 = xb * ir_ref[b, 0, 0] + yb * ir_ref[b, 0, 1] + zb * ir_ref[b, 0, 2]
        s1 = xb * ir_ref[b, 1, 0] + yb * ir_ref[b, 1, 1] + zb * ir_ref[b, 1, 2]
        s2 = xb * ir_ref[b, 2, 0] + yb * ir_ref[b, 2, 1] + zb * ir_ref[b, 2, 2]
        sb0, sb1, sb2 = _bf(s0), _bf(s1), _bf(s2)
        for c in range(NC):
            p0 = tt_ref[b, c, 0] + sb0 * ab_ref[b, c, 0, 0] + sb1 * ab_ref[b, c, 0, 1] + sb2 * ab_ref[b, c, 0, 2]
            p1 = tt_ref[b, c, 1] + sb0 * ab_ref[b, c, 1, 0] + sb1 * ab_ref[b, c, 1, 1] + sb2 * ab_ref[b, c, 1, 2]
            p2 = tt_ref[b, c, 2] + sb0 * ab_ref[b, c, 2, 0] + sb1 * ab_ref[b, c, 2, 1] + sb2 * ab_ref[b, c, 2, 2]
            u = p0 / p2
            v = p1 / p2
            ub, vb, db = _bf(u), _bf(v), _bf(p2)
            q0 = pt_ref[b, c, 0] + ub * pr_ref[b, c, 0, 0] + vb * pr_ref[b, c, 0, 1] + db * pr_ref[b, c, 0, 2]
            q1 = pt_ref[b, c, 1] + ub * pr_ref[b, c, 1, 0] + vb * pr_ref[b, c, 1, 1] + db * pr_ref[b, c, 1, 2]
            q2 = pt_ref[b, c, 2] + ub * pr_ref[b, c, 2, 0] + vb * pr_ref[b, c, 2, 1] + db * pr_ref[b, c, 2, 2]
            cx = jnp.round(q0 / float(DS))
            cy = jnp.round(q1 / float(DS))
            kept = ((cx >= 0) & (cx < W_F) & (cy >= 0) & (cy < H_F)
                    & (q2 < D_MAX) & (q2 >= D_MIN))
            cxi = jnp.clip(jnp.where(jnp.isnan(cx), 0.0, cx), 0.0, W_F - 1.0).astype(jnp.int32)
            cyi = jnp.clip(jnp.where(jnp.isnan(cy), 0.0, cy), 0.0, H_F - 1.0).astype(jnp.int32)
            g = c * PIX + cyi * W_F + cxi
            out_ref[c, sl] = jnp.where(kept, g, LOC_ZERO)


def _project_indices(xs, ys, zs, bt, ir, ab, tt, pr, pt):
    return pl.pallas_call(
        _proj_body,
        in_specs=[pl.BlockSpec(memory_space=pltpu.VMEM)] * 3
        + [pl.BlockSpec(memory_space=pltpu.SMEM)] * 6,
        out_specs=pl.BlockSpec(memory_space=pltpu.VMEM),
        out_shape=jax.ShapeDtypeStruct((NC, N_PTS), jnp.int32),
    )(xs, ys, zs, bt, ir, ab, tt, pr, pt)


@functools.cache
def _make_sc_gather_acc():
    return functools.partial(
        pl.kernel,
        out_type=jax.ShapeDtypeStruct((D_OUT, N_PAD), jnp.float32),
        mesh=plsc.VectorSubcoreMesh(core_axis_name="c", subcore_axis_name="s"),
        scratch_types=[
            pltpu.VMEM((LOC_ROWS * DSLICE,), jnp.float32),
            pltpu.VMEM((NC, CH), jnp.int32),
            pltpu.VMEM((DSLICE, CH), jnp.float32),
        ],
        compiler_params=pltpu.CompilerParams(needs_layout_passes=False),
    )(_sc_body)


def _sc_body(table_hbm, gidx_hbm, out_hbm, tbl_v, idx_v, outs_v):
    wid = lax.axis_index("s") * 2 + lax.axis_index("c")
    dg = wid % NDG
    pg = wid // NDG
    b = wid // (NDG * 2)
    pltpu.sync_copy(table_hbm.at[b, dg], tbl_v)
    pt0 = pg * PG_PTS

    def chunk_body(k, carry):
        base = pt0 + k * CH
        for c in range(NC):
            pltpu.sync_copy(gidx_hbm.at[c, pl.ds(base, CH)], idx_v.at[c])

        def group_body(g, carry2):
            gsl = pl.ds(g * 16, 16)
            accs = [None] * DSLICE
            for c in range(NC):
                flat0 = idx_v[c, gsl] * DSLICE
                for d in range(DSLICE):
                    v = plsc.load_gather(tbl_v, [flat0 + d])
                    accs[d] = v if c == 0 else accs[d] + v
            for d in range(DSLICE):
                outs_v[d, gsl] = accs[d]
            return carry2

        lax.fori_loop(0, GROUPS, group_body, 0)
        pltpu.sync_copy(outs_v,
                        out_hbm.at[pl.ds(dg * DSLICE, DSLICE), pl.ds(base, CH)])
        return carry

    lax.fori_loop(0, NCHUNK, chunk_body, 0)


def kernel(voxel_features, voxel_coords, img_feats, rots, trans, intrins,
           post_rots, post_trans, bda, lidar2cam, W, imgs):
    f32 = jnp.float32
    bf16 = jnp.bfloat16
    eye4 = jnp.eye(4, dtype=f32)
    c2i = jnp.tile(eye4, (BS, NC, 1, 1))
    c2i = c2i.at[:, :, :3, :3].set(intrins)
    l2i = jnp.einsum("bcij,bckj->bcik", c2i, lidar2cam)
    ab = l2i[:, :, :3, :3].astype(bf16).astype(f32)
    tt = l2i[:, :, :3, 3]
    ir = jnp.linalg.inv(bda[:, :3, :3]).astype(bf16).astype(f32)
    bt = bda[:, :3, 3]
    prb = post_rots.astype(f32).astype(bf16).astype(f32)
    ptf = post_trans.astype(f32)

    xs = voxel_coords[:, 3].astype(f32)
    ys = voxel_coords[:, 2].astype(f32)
    zs = voxel_coords[:, 1].astype(f32)

    feats2d = img_feats.reshape(BS * NC, C_IMG, PIX).astype(bf16).astype(f32)
    t12 = _build_tables(feats2d, W.T.astype(bf16).astype(f32))
    tb = t12.reshape(BS, NC * PIX, D_OUT)
    tb = jnp.concatenate(
        [tb, jnp.zeros((BS, LOC_ROWS - LOC_ZERO, D_OUT), f32)], axis=1)
    table = (tb.reshape(BS, LOC_ROWS, NDG, DSLICE)
             .transpose(0, 2, 1, 3)
             .reshape(BS, NDG, LOC_ROWS * DSLICE))

    gidx = _project_indices(xs, ys, zs, bt, ir, ab, tt, prb, ptf)
    padc = jnp.full((NC, 2 * PG_PTS - NPB), LOC_ZERO, jnp.int32)
    gidx_pad = jnp.concatenate(
        [gidx[:, :NPB], padc, gidx[:, NPB:], padc], axis=1)

    img_t = _make_sc_gather_acc()(table, gidx_pad)
    img_all = img_t.T
    img_voxel = jnp.concatenate(
        [img_all[:NPB], img_all[2 * PG_PTS:2 * PG_PTS + NPB]], axis=0)

    out_features = jnp.concatenate([voxel_features, img_voxel], axis=0)
    out_coords = jnp.concatenate([voxel_coords, voxel_coords], axis=0)
    return (out_features, out_coords)


# speedup vs baseline: 45.1428x; 1.0780x over previous
import functools

import jax
import jax.numpy as jnp
import numpy as np
from jax import lax
from jax.experimental import pallas as pl
from jax.experimental.pallas import tpu as pltpu
from jax.experimental.pallas import tpu_sc as plsc

BS = 2
NC = 6
NPB = 50000
C_IMG = 256
D_OUT = 128
H_F = 16
W_F = 44
DS = 16
N_PTS = BS * NPB
PIX = H_F * W_F
LOC_ZERO = NC * PIX
LOC_ROWS = LOC_ZERO + 8
DSLICE = 16
NDG = D_OUT // DSLICE
NPG = 4
PG_PTS = 25600
N_PAD = NPG * PG_PTS
CH = 1280
NCHUNK = PG_PTS // CH
GROUPS = CH // 16
VOXEL_SIZE = np.array([0.1, 0.1, 0.2], dtype=np.float32)
PC_RANGE = np.array([-51.2, -51.2, -5.0], dtype=np.float32)
D_MIN, D_MAX = 1.0, 60.0


def _table_body(f_ref, w_ref, o_ref):
    wb = w_ref[...].astype(jnp.bfloat16)
    for c in range(NC):
        o_ref[0, 0, :, pl.ds(c * PIX, PIX)] = lax.dot_general(
            wb, f_ref[0, c].astype(jnp.bfloat16),
            dimension_numbers=(((1,), (0,)), ((), ())),
            preferred_element_type=jnp.float32,
        )


def _build_tables(feats4d, w):
    return pl.pallas_call(
        _table_body,
        grid=(BS, NDG),
        in_specs=[
            pl.BlockSpec((1, NC, C_IMG, PIX), lambda g, h: (g, 0, 0, 0)),
            pl.BlockSpec((DSLICE, C_IMG), lambda g, h: (h, 0)),
        ],
        out_specs=pl.BlockSpec(
            (1, 1, DSLICE, NC * PIX), lambda g, h: (g, h, 0, 0)),
        out_shape=jax.ShapeDtypeStruct((BS, NDG, DSLICE, NC * PIX),
                                       jnp.float32),
    )(feats4d, w)


def _bf(x):
    return x.astype(jnp.bfloat16).astype(jnp.float32)


def _proj_body(x_ref, y_ref, z_ref, bt_ref, ir_ref, ab_ref, tt_ref, pr_ref,
               pt_ref, out_ref):
    for b in range(BS):
        sl = pl.ds(b * NPB, NPB)
        x1 = (x_ref[sl] * float(VOXEL_SIZE[0]) + float(PC_RANGE[0])) - bt_ref[b, 0]
        y1 = (y_ref[sl] * float(VOXEL_SIZE[1]) + float(PC_RANGE[1])) - bt_ref[b, 1]
        z1 = (z_ref[sl] * float(VOXEL_SIZE[2]) + float(PC_RANGE[2])) - bt_ref[b, 2]
        xb, yb, zb = _bf(x1), _bf(y1), _bf(z1)
        s0 = xb * ir_ref[b, 0, 0] + yb * ir_ref[b, 0, 1] + zb * ir_ref[b, 0, 2]
        s1 = xb * ir_ref[b, 1, 0] + yb * ir_ref[b, 1, 1] + zb * ir_ref[b, 1, 2]
        s2 = xb * ir_ref[b, 2, 0] + yb * ir_ref[b, 2, 1] + zb * ir_ref[b, 2, 2]
        sb0, sb1, sb2 = _bf(s0), _bf(s1), _bf(s2)
        for c in range(NC):
            p0 = tt_ref[b, c, 0] + sb0 * ab_ref[b, c, 0, 0] + sb1 * ab_ref[b, c, 0, 1] + sb2 * ab_ref[b, c, 0, 2]
            p1 = tt_ref[b, c, 1] + sb0 * ab_ref[b, c, 1, 0] + sb1 * ab_ref[b, c, 1, 1] + sb2 * ab_ref[b, c, 1, 2]
            p2 = tt_ref[b, c, 2] + sb0 * ab_ref[b, c, 2, 0] + sb1 * ab_ref[b, c, 2, 1] + sb2 * ab_ref[b, c, 2, 2]
            u = p0 / p2
            v = p1 / p2
            ub, vb, db = _bf(u), _bf(v), _bf(p2)
            q0 = pt_ref[b, c, 0] + ub * pr_ref[b, c, 0, 0] + vb * pr_ref[b, c, 0, 1] + db * pr_ref[b, c, 0, 2]
            q1 = pt_ref[b, c, 1] + ub * pr_ref[b, c, 1, 0] + vb * pr_ref[b, c, 1, 1] + db * pr_ref[b, c, 1, 2]
            q2 = pt_ref[b, c, 2] + ub * pr_ref[b, c, 2, 0] + vb * pr_ref[b, c, 2, 1] + db * pr_ref[b, c, 2, 2]
            cx = jnp.round(q0 / float(DS))
            cy = jnp.round(q1 / float(DS))
            kept = ((cx >= 0) & (cx < W_F) & (cy >= 0) & (cy < H_F)
                    & (q2 < D_MAX) & (q2 >= D_MIN))
            cxi = jnp.clip(jnp.where(jnp.isnan(cx), 0.0, cx), 0.0, W_F - 1.0).astype(jnp.int32)
            cyi = jnp.clip(jnp.where(jnp.isnan(cy), 0.0, cy), 0.0, H_F - 1.0).astype(jnp.int32)
            g = c * PIX + cyi * W_F + cxi
            out_ref[c, pl.ds(b * 2 * PG_PTS, NPB)] = jnp.where(kept, g, LOC_ZERO)
    padv = jnp.full((2 * PG_PTS - NPB,), LOC_ZERO, jnp.int32)
    for c in range(NC):
        out_ref[c, pl.ds(NPB, 2 * PG_PTS - NPB)] = padv
        out_ref[c, pl.ds(2 * PG_PTS + NPB, 2 * PG_PTS - NPB)] = padv


def _project_indices(xs, ys, zs, bt, ir, ab, tt, pr, pt):
    return pl.pallas_call(
        _proj_body,
        in_specs=[pl.BlockSpec(memory_space=pltpu.VMEM)] * 3
        + [pl.BlockSpec(memory_space=pltpu.SMEM)] * 6,
        out_specs=pl.BlockSpec(memory_space=pltpu.VMEM),
        out_shape=jax.ShapeDtypeStruct((NC, N_PAD), jnp.int32),
    )(xs, ys, zs, bt, ir, ab, tt, pr, pt)


@functools.cache
def _make_sc_gather_acc():
    return functools.partial(
        pl.kernel,
        out_type=jax.ShapeDtypeStruct((D_OUT, N_PAD), jnp.float32),
        mesh=plsc.VectorSubcoreMesh(core_axis_name="c", subcore_axis_name="s"),
        scratch_types=[
            pltpu.VMEM((DSLICE, LOC_ZERO + 16), jnp.float32),
            pltpu.VMEM((NC, CH), jnp.int32),
            pltpu.VMEM((DSLICE, CH), jnp.float32),
        ],
        compiler_params=pltpu.CompilerParams(needs_layout_passes=False),
    )(_sc_body)


def _sc_body(table_hbm, gidx_hbm, out_hbm, tbl_v, idx_v, outs_v):
    wid = lax.axis_index("s") * 2 + lax.axis_index("c")
    dg = wid % NDG
    pg = wid // NDG
    b = wid // (NDG * 2)
    pltpu.sync_copy(table_hbm.at[b, dg], tbl_v.at[:, pl.ds(0, NC * PIX)])
    zeros16 = jnp.zeros((16,), jnp.float32)
    for r in range(DSLICE):
        tbl_v[r, pl.ds(LOC_ZERO, 16)] = zeros16
    pt0 = pg * PG_PTS

    def chunk_body(k, carry):
        base = pt0 + k * CH
        for c in range(NC):
            pltpu.sync_copy(gidx_hbm.at[c, pl.ds(base, CH)], idx_v.at[c])

        def group_body(g, carry2):
            gsl = pl.ds(g * 16, 16)
            accs = [None] * DSLICE
            for c in range(NC):
                rows = idx_v[c, gsl]
                for d in range(DSLICE):
                    v = plsc.load_gather(
                        tbl_v, [jnp.full((16,), d, jnp.int32), rows])
                    accs[d] = v if c == 0 else accs[d] + v
            for d in range(DSLICE):
                outs_v[d, gsl] = accs[d]
            return carry2

        lax.fori_loop(0, GROUPS, group_body, 0)
        pltpu.sync_copy(outs_v,
                        out_hbm.at[pl.ds(dg * DSLICE, DSLICE), pl.ds(base, CH)])
        return carry

    lax.fori_loop(0, NCHUNK, chunk_body, 0)


def kernel(voxel_features, voxel_coords, img_feats, rots, trans, intrins,
           post_rots, post_trans, bda, lidar2cam, W, imgs):
    f32 = jnp.float32
    bf16 = jnp.bfloat16
    eye4 = jnp.eye(4, dtype=f32)
    c2i = jnp.tile(eye4, (BS, NC, 1, 1))
    c2i = c2i.at[:, :, :3, :3].set(intrins)
    l2i = jnp.einsum("bcij,bckj->bcik", c2i, lidar2cam)
    ab = l2i[:, :, :3, :3].astype(bf16).astype(f32)
    tt = l2i[:, :, :3, 3]
    ir = jnp.linalg.inv(bda[:, :3, :3]).astype(bf16).astype(f32)
    bt = bda[:, :3, 3]
    prb = post_rots.astype(f32).astype(bf16).astype(f32)
    ptf = post_trans.astype(f32)

    xs = voxel_coords[:, 3].astype(f32)
    ys = voxel_coords[:, 2].astype(f32)
    zs = voxel_coords[:, 1].astype(f32)

    table = _build_tables(img_feats.reshape(BS, NC, C_IMG, PIX), W)

    gidx_pad = _project_indices(xs, ys, zs, bt, ir, ab, tt, prb, ptf)

    img_t = _make_sc_gather_acc()(table, gidx_pad)
    img_all = img_t.T
    img_voxel = jnp.concatenate(
        [img_all[:NPB], img_all[2 * PG_PTS:2 * PG_PTS + NPB]], axis=0)

    out_features = jnp.concatenate([voxel_features, img_voxel], axis=0)
    out_coords = jnp.concatenate([voxel_coords, voxel_coords], axis=0)
    return (out_features, out_coords)


# speedup vs baseline: 47.8933x; 1.0609x over previous
import functools

import jax
import jax.numpy as jnp
import numpy as np
from jax import lax
from jax.experimental import pallas as pl
from jax.experimental.pallas import tpu as pltpu
from jax.experimental.pallas import tpu_sc as plsc

BS = 2
NC = 6
NPB = 50000
C_IMG = 256
D_OUT = 128
H_F = 16
W_F = 44
DS = 16
N_PTS = BS * NPB
PIX = H_F * W_F
LOC_ZERO = NC * PIX
LOC_ROWS = LOC_ZERO + 8
DSLICE = 16
NDG = D_OUT // DSLICE
NPG = 4
PG_PTS = 25600
N_PAD = NPG * PG_PTS
PROW = 12500
PROWP = 12800
CH = 1280
NCHUNK = PG_PTS // CH
GROUPS = CH // 16
VOXEL_SIZE = np.array([0.1, 0.1, 0.2], dtype=np.float32)
PC_RANGE = np.array([-51.2, -51.2, -5.0], dtype=np.float32)
D_MIN, D_MAX = 1.0, 60.0


def _table_body(f_ref, w_ref, o_ref):
    wb = w_ref[...].astype(jnp.bfloat16)
    for c in range(NC):
        o_ref[0, 0, :, pl.ds(c * PIX, PIX)] = lax.dot_general(
            wb, f_ref[0, c].astype(jnp.bfloat16),
            dimension_numbers=(((1,), (0,)), ((), ())),
            preferred_element_type=jnp.float32,
        )


def _build_tables(feats4d, w):
    return pl.pallas_call(
        _table_body,
        grid=(BS, NDG),
        in_specs=[
            pl.BlockSpec((1, NC, C_IMG, PIX), lambda g, h: (g, 0, 0, 0)),
            pl.BlockSpec((DSLICE, C_IMG), lambda g, h: (h, 0)),
        ],
        out_specs=pl.BlockSpec(
            (1, 1, DSLICE, NC * PIX), lambda g, h: (g, h, 0, 0)),
        out_shape=jax.ShapeDtypeStruct((BS, NDG, DSLICE, NC * PIX),
                                       jnp.float32),
    )(feats4d, w)


def _bf(x):
    return x.astype(jnp.bfloat16).astype(jnp.float32)


def _proj_body(x_ref, y_ref, z_ref, bt_ref, ir_ref, ab_ref, tt_ref, pr_ref,
               pt_ref, out_ref):
    def col(ref, *idx):
        return ref[idx[0], :, pl.ds(idx[1], 1)] if len(idx) == 2 else \
            ref[:, pl.ds(idx[0], 1)]

    x = x_ref[...]
    y = y_ref[...]
    z = z_ref[...]
    x1 = (x * float(VOXEL_SIZE[0]) + float(PC_RANGE[0])) - col(bt_ref, 0)
    y1 = (y * float(VOXEL_SIZE[1]) + float(PC_RANGE[1])) - col(bt_ref, 1)
    z1 = (z * float(VOXEL_SIZE[2]) + float(PC_RANGE[2])) - col(bt_ref, 2)
    xb, yb, zb = _bf(x1), _bf(y1), _bf(z1)
    s0 = xb * col(ir_ref, 0) + yb * col(ir_ref, 1) + zb * col(ir_ref, 2)
    s1 = xb * col(ir_ref, 3) + yb * col(ir_ref, 4) + zb * col(ir_ref, 5)
    s2 = xb * col(ir_ref, 6) + yb * col(ir_ref, 7) + zb * col(ir_ref, 8)
    sb0, sb1, sb2 = _bf(s0), _bf(s1), _bf(s2)
    for c in range(NC):
        p0 = col(tt_ref, c, 0) + sb0 * col(ab_ref, c, 0) + sb1 * col(ab_ref, c, 1) + sb2 * col(ab_ref, c, 2)
        p1 = col(tt_ref, c, 1) + sb0 * col(ab_ref, c, 3) + sb1 * col(ab_ref, c, 4) + sb2 * col(ab_ref, c, 5)
        p2 = col(tt_ref, c, 2) + sb0 * col(ab_ref, c, 6) + sb1 * col(ab_ref, c, 7) + sb2 * col(ab_ref, c, 8)
        u = p0 / p2
        v = p1 / p2
        ub, vb, db = _bf(u), _bf(v), _bf(p2)
        q0 = col(pt_ref, c, 0) + ub * col(pr_ref, c, 0) + vb * col(pr_ref, c, 1) + db * col(pr_ref, c, 2)
        q1 = col(pt_ref, c, 1) + ub * col(pr_ref, c, 3) + vb * col(pr_ref, c, 4) + db * col(pr_ref, c, 5)
        q2 = col(pt_ref, c, 2) + ub * col(pr_ref, c, 6) + vb * col(pr_ref, c, 7) + db * col(pr_ref, c, 8)
        cx = jnp.round(q0 / float(DS))
        cy = jnp.round(q1 / float(DS))
        kept = ((cx >= 0) & (cx < W_F) & (cy >= 0) & (cy < H_F)
                & (q2 < D_MAX) & (q2 >= D_MIN))
        cxi = jnp.clip(jnp.where(jnp.isnan(cx), 0.0, cx), 0.0, W_F - 1.0).astype(jnp.int32)
        cyi = jnp.clip(jnp.where(jnp.isnan(cy), 0.0, cy), 0.0, H_F - 1.0).astype(jnp.int32)
        g = c * PIX + cyi * W_F + cxi
        out_ref[c, :, pl.ds(0, PROW)] = jnp.where(kept, g, LOC_ZERO)
        out_ref[c, :, pl.ds(PROW, PROWP - PROW)] = jnp.full(
            (8, PROWP - PROW), LOC_ZERO, jnp.int32)


def _project_indices(xs, ys, zs, bt, ir, ab, tt, pr, pt):
    return pl.pallas_call(
        _proj_body,
        in_specs=[pl.BlockSpec(memory_space=pltpu.VMEM)] * 9,
        out_specs=pl.BlockSpec(memory_space=pltpu.VMEM),
        out_shape=jax.ShapeDtypeStruct((NC, 8, PROWP), jnp.int32),
    )(xs, ys, zs, bt, ir, ab, tt, pr, pt)


@functools.cache
def _make_sc_gather_acc():
    return functools.partial(
        pl.kernel,
        out_type=jax.ShapeDtypeStruct((D_OUT, N_PAD), jnp.float32),
        mesh=plsc.VectorSubcoreMesh(core_axis_name="c", subcore_axis_name="s"),
        scratch_types=[
            pltpu.VMEM((DSLICE, LOC_ZERO + 16), jnp.float32),
            pltpu.VMEM((NC, CH), jnp.int32),
            pltpu.VMEM((DSLICE, CH), jnp.float32),
        ],
        compiler_params=pltpu.CompilerParams(needs_layout_passes=False),
    )(_sc_body)


def _sc_body(table_hbm, gidx_hbm, out_hbm, tbl_v, idx_v, outs_v):
    wid = lax.axis_index("s") * 2 + lax.axis_index("c")
    dg = wid % NDG
    pg = wid // NDG
    b = wid // (NDG * 2)
    pltpu.sync_copy(table_hbm.at[b, dg], tbl_v.at[:, pl.ds(0, NC * PIX)])
    zeros16 = jnp.zeros((16,), jnp.float32)
    for r in range(DSLICE):
        tbl_v[r, pl.ds(LOC_ZERO, 16)] = zeros16
    pt0 = pg * PG_PTS

    def chunk_body(k, carry):
        base = pt0 + k * CH
        for c in range(NC):
            pltpu.sync_copy(gidx_hbm.at[c, pl.ds(base, CH)], idx_v.at[c])

        def group_body(g, carry2):
            gsl = pl.ds(g * 16, 16)
            accs = [None] * DSLICE
            for c in range(NC):
                rows = idx_v[c, gsl]
                for d in range(DSLICE):
                    v = plsc.load_gather(
                        tbl_v, [jnp.full((16,), d, jnp.int32), rows])
                    accs[d] = v if c == 0 else accs[d] + v
            for d in range(DSLICE):
                outs_v[d, gsl] = accs[d]
            return carry2

        lax.fori_loop(0, GROUPS, group_body, 0)
        pltpu.sync_copy(outs_v,
                        out_hbm.at[pl.ds(dg * DSLICE, DSLICE), pl.ds(base, CH)])
        return carry

    lax.fori_loop(0, NCHUNK, chunk_body, 0)


def kernel(voxel_features, voxel_coords, img_feats, rots, trans, intrins,
           post_rots, post_trans, bda, lidar2cam, W, imgs):
    f32 = jnp.float32
    bf16 = jnp.bfloat16
    eye4 = jnp.eye(4, dtype=f32)
    c2i = jnp.tile(eye4, (BS, NC, 1, 1))
    c2i = c2i.at[:, :, :3, :3].set(intrins)
    l2i = jnp.einsum("bcij,bckj->bcik", c2i, lidar2cam)
    ab = l2i[:, :, :3, :3].astype(bf16).astype(f32)
    tt = l2i[:, :, :3, 3]
    ir = jnp.linalg.inv(bda[:, :3, :3]).astype(bf16).astype(f32)
    bt = bda[:, :3, 3]
    prb = post_rots.astype(f32).astype(bf16).astype(f32)
    ptf = post_trans.astype(f32)
    bt8 = jnp.repeat(bt, 4, axis=0)
    ir8 = jnp.repeat(ir.reshape(BS, 9), 4, axis=0)
    ab8 = jnp.repeat(ab.transpose(1, 0, 2, 3).reshape(NC, BS, 9), 4, axis=1)
    tt8 = jnp.repeat(tt.transpose(1, 0, 2), 4, axis=1)
    pr8 = jnp.repeat(prb.transpose(1, 0, 2, 3).reshape(NC, BS, 9), 4, axis=1)
    pt8 = jnp.repeat(ptf.transpose(1, 0, 2), 4, axis=1)

    xs = voxel_coords[:, 3].astype(f32).reshape(8, PROW)
    ys = voxel_coords[:, 2].astype(f32).reshape(8, PROW)
    zs = voxel_coords[:, 1].astype(f32).reshape(8, PROW)

    table = _build_tables(img_feats.reshape(BS, NC, C_IMG, PIX), W)

    gidx_pad = _project_indices(xs, ys, zs, bt8, ir8, ab8, tt8, pr8, pt8)

    img_t = _make_sc_gather_acc()(table, gidx_pad.reshape(NC, N_PAD))
    img_voxel = (img_t.reshape(D_OUT, 8, PROWP)[:, :, :PROW]
                 .reshape(D_OUT, N_PTS).T)

    out_features = jnp.concatenate([voxel_features, img_voxel], axis=0)
    out_coords = jnp.concatenate([voxel_coords, voxel_coords], axis=0)
    return (out_features, out_coords)


# speedup vs baseline: 58.0756x; 1.2126x over previous
import functools

import jax
import jax.numpy as jnp
import numpy as np
from jax import lax
from jax.experimental import pallas as pl
from jax.experimental.pallas import tpu as pltpu
from jax.experimental.pallas import tpu_sc as plsc

BS = 2
NC = 6
NPB = 50000
C_IMG = 256
D_OUT = 128
H_F = 16
W_F = 44
DS = 16
N_PTS = BS * NPB
PIX = H_F * W_F
LOC_ZERO = NC * PIX
LOC_ROWS = LOC_ZERO + 8
DSLICE = 16
NDG = D_OUT // DSLICE
NPG = 4
PG_PTS = 25600
N_PAD = NPG * PG_PTS
PROW = 12500
PROWP = 12800
CH = 1024
NCHUNK = PG_PTS // CH
GROUPS = CH // 16
VOXEL_SIZE = np.array([0.1, 0.1, 0.2], dtype=np.float32)
PC_RANGE = np.array([-51.2, -51.2, -5.0], dtype=np.float32)
D_MIN, D_MAX = 1.0, 60.0


def _table_body(f_ref, w_ref, o_ref):
    wb = w_ref[...].astype(jnp.bfloat16)
    for c in range(NC):
        o_ref[0, 0, :, pl.ds(c * PIX, PIX)] = lax.dot_general(
            wb, f_ref[0, c].astype(jnp.bfloat16),
            dimension_numbers=(((1,), (0,)), ((), ())),
            preferred_element_type=jnp.float32,
        )


def _build_tables(feats4d, w):
    return pl.pallas_call(
        _table_body,
        grid=(BS, NDG),
        in_specs=[
            pl.BlockSpec((1, NC, C_IMG, PIX), lambda g, h: (g, 0, 0, 0)),
            pl.BlockSpec((DSLICE, C_IMG), lambda g, h: (h, 0)),
        ],
        out_specs=pl.BlockSpec(
            (1, 1, DSLICE, NC * PIX), lambda g, h: (g, h, 0, 0)),
        out_shape=jax.ShapeDtypeStruct((BS, NDG, DSLICE, NC * PIX),
                                       jnp.float32),
    )(feats4d, w)


def _bf(x):
    return x.astype(jnp.bfloat16).astype(jnp.float32)


def _proj_body(x_ref, y_ref, z_ref, bt_ref, ir_ref, ab_ref, tt_ref, pr_ref,
               pt_ref, out_ref):
    def col(ref, *idx):
        return ref[idx[0], :, pl.ds(idx[1], 1)] if len(idx) == 2 else \
            ref[:, pl.ds(idx[0], 1)]

    x = x_ref[...]
    y = y_ref[...]
    z = z_ref[...]
    x1 = (x * float(VOXEL_SIZE[0]) + float(PC_RANGE[0])) - col(bt_ref, 0)
    y1 = (y * float(VOXEL_SIZE[1]) + float(PC_RANGE[1])) - col(bt_ref, 1)
    z1 = (z * float(VOXEL_SIZE[2]) + float(PC_RANGE[2])) - col(bt_ref, 2)
    xb, yb, zb = _bf(x1), _bf(y1), _bf(z1)
    s0 = xb * col(ir_ref, 0) + yb * col(ir_ref, 1) + zb * col(ir_ref, 2)
    s1 = xb * col(ir_ref, 3) + yb * col(ir_ref, 4) + zb * col(ir_ref, 5)
    s2 = xb * col(ir_ref, 6) + yb * col(ir_ref, 7) + zb * col(ir_ref, 8)
    sb0, sb1, sb2 = _bf(s0), _bf(s1), _bf(s2)
    for c in range(NC):
        p0 = col(tt_ref, c, 0) + sb0 * col(ab_ref, c, 0) + sb1 * col(ab_ref, c, 1) + sb2 * col(ab_ref, c, 2)
        p1 = col(tt_ref, c, 1) + sb0 * col(ab_ref, c, 3) + sb1 * col(ab_ref, c, 4) + sb2 * col(ab_ref, c, 5)
        p2 = col(tt_ref, c, 2) + sb0 * col(ab_ref, c, 6) + sb1 * col(ab_ref, c, 7) + sb2 * col(ab_ref, c, 8)
        u = p0 / p2
        v = p1 / p2
        ub, vb, db = _bf(u), _bf(v), _bf(p2)
        q0 = col(pt_ref, c, 0) + ub * col(pr_ref, c, 0) + vb * col(pr_ref, c, 1) + db * col(pr_ref, c, 2)
        q1 = col(pt_ref, c, 1) + ub * col(pr_ref, c, 3) + vb * col(pr_ref, c, 4) + db * col(pr_ref, c, 5)
        q2 = col(pt_ref, c, 2) + ub * col(pr_ref, c, 6) + vb * col(pr_ref, c, 7) + db * col(pr_ref, c, 8)
        cx = jnp.round(q0 / float(DS))
        cy = jnp.round(q1 / float(DS))
        kept = ((cx >= 0) & (cx < W_F) & (cy >= 0) & (cy < H_F)
                & (q2 < D_MAX) & (q2 >= D_MIN))
        cxi = jnp.clip(jnp.where(jnp.isnan(cx), 0.0, cx), 0.0, W_F - 1.0).astype(jnp.int32)
        cyi = jnp.clip(jnp.where(jnp.isnan(cy), 0.0, cy), 0.0, H_F - 1.0).astype(jnp.int32)
        g = c * PIX + cyi * W_F + cxi
        out_ref[c, :, pl.ds(0, PROW)] = jnp.where(kept, g, LOC_ZERO)
        out_ref[c, :, pl.ds(PROW, PROWP - PROW)] = jnp.full(
            (8, PROWP - PROW), LOC_ZERO, jnp.int32)


def _project_indices(xs, ys, zs, bt, ir, ab, tt, pr, pt):
    return pl.pallas_call(
        _proj_body,
        in_specs=[pl.BlockSpec(memory_space=pltpu.VMEM)] * 9,
        out_specs=pl.BlockSpec(memory_space=pltpu.VMEM),
        out_shape=jax.ShapeDtypeStruct((NC, 8, PROWP), jnp.int32),
    )(xs, ys, zs, bt, ir, ab, tt, pr, pt)


@functools.cache
def _make_sc_gather_acc():
    return functools.partial(
        pl.kernel,
        out_type=jax.ShapeDtypeStruct((D_OUT, N_PAD), jnp.float32),
        mesh=plsc.VectorSubcoreMesh(core_axis_name="c", subcore_axis_name="s"),
        scratch_types=[
            pltpu.VMEM((DSLICE, LOC_ZERO + 16), jnp.float32),
            pltpu.VMEM((2, NC, CH), jnp.int32),
            pltpu.VMEM((2, DSLICE, CH), jnp.float32),
            pltpu.SemaphoreType.DMA,
            pltpu.SemaphoreType.DMA,
        ],
        compiler_params=pltpu.CompilerParams(needs_layout_passes=False),
    )(_sc_body)


def _sc_body(table_hbm, gidx_hbm, out_hbm, tbl_v, idx_v, outs_v, sem_i, sem_o):
    wid = lax.axis_index("s") * 2 + lax.axis_index("c")
    dg = wid % NDG
    pg = wid // NDG
    b = wid // (NDG * 2)
    pltpu.sync_copy(table_hbm.at[b, dg], tbl_v.at[:, pl.ds(0, NC * PIX)])
    zeros16 = jnp.zeros((16,), jnp.float32)
    for r in range(DSLICE):
        tbl_v[r, pl.ds(LOC_ZERO, 16)] = zeros16
    pt0 = pg * PG_PTS

    def fetch(k, buf):
        return pltpu.async_copy(
            gidx_hbm.at[:, pl.ds(pt0 + k * CH, CH)], idx_v.at[buf], sem_i)

    def make_group_body(buf):
        def group_body(g, carry2):
            gsl = pl.ds(g * 16, 16)
            accs = [None] * DSLICE
            for c in range(NC):
                rows = idx_v[buf, c, gsl]
                for d in range(DSLICE):
                    v = plsc.load_gather(
                        tbl_v, [jnp.full((16,), d, jnp.int32), rows])
                    accs[d] = v if c == 0 else accs[d] + v
            for d in range(DSLICE):
                outs_v[buf, d, gsl] = accs[d]
            return carry2
        return group_body

    write_handles = [None] * NCHUNK
    fetch_handles = [None] * NCHUNK
    fetch_handles[0] = fetch(0, 0)
    for k in range(NCHUNK):
        buf = k % 2
        fetch_handles[k].wait()
        if k + 1 < NCHUNK:
            fetch_handles[k + 1] = fetch(k + 1, 1 - buf)
        if k >= 2:
            write_handles[k - 2].wait()
        lax.fori_loop(0, GROUPS, make_group_body(buf), 0)
        write_handles[k] = pltpu.async_copy(
            outs_v.at[buf],
            out_hbm.at[pl.ds(dg * DSLICE, DSLICE), pl.ds(pt0 + k * CH, CH)],
            sem_o)
    write_handles[NCHUNK - 2].wait()
    write_handles[NCHUNK - 1].wait()


def kernel(voxel_features, voxel_coords, img_feats, rots, trans, intrins,
           post_rots, post_trans, bda, lidar2cam, W, imgs):
    f32 = jnp.float32
    bf16 = jnp.bfloat16
    eye4 = jnp.eye(4, dtype=f32)
    c2i = jnp.tile(eye4, (BS, NC, 1, 1))
    c2i = c2i.at[:, :, :3, :3].set(intrins)
    l2i = jnp.einsum("bcij,bckj->bcik", c2i, lidar2cam)
    ab = l2i[:, :, :3, :3].astype(bf16).astype(f32)
    tt = l2i[:, :, :3, 3]
    ir = jnp.linalg.inv(bda[:, :3, :3]).astype(bf16).astype(f32)
    bt = bda[:, :3, 3]
    prb = post_rots.astype(f32).astype(bf16).astype(f32)
    ptf = post_trans.astype(f32)
    bt8 = jnp.repeat(bt, 4, axis=0)
    ir8 = jnp.repeat(ir.reshape(BS, 9), 4, axis=0)
    ab8 = jnp.repeat(ab.transpose(1, 0, 2, 3).reshape(NC, BS, 9), 4, axis=1)
    tt8 = jnp.repeat(tt.transpose(1, 0, 2), 4, axis=1)
    pr8 = jnp.repeat(prb.transpose(1, 0, 2, 3).reshape(NC, BS, 9), 4, axis=1)
    pt8 = jnp.repeat(ptf.transpose(1, 0, 2), 4, axis=1)

    xs = voxel_coords[:, 3].astype(f32).reshape(8, PROW)
    ys = voxel_coords[:, 2].astype(f32).reshape(8, PROW)
    zs = voxel_coords[:, 1].astype(f32).reshape(8, PROW)

    table = _build_tables(img_feats.reshape(BS, NC, C_IMG, PIX), W)

    gidx_pad = _project_indices(xs, ys, zs, bt8, ir8, ab8, tt8, pr8, pt8)

    img_t = _make_sc_gather_acc()(table, gidx_pad.reshape(NC, N_PAD))
    img_voxel = (img_t.reshape(D_OUT, 8, PROWP)[:, :, :PROW]
                 .reshape(D_OUT, N_PTS).T)

    out_features = jnp.concatenate([voxel_features, img_voxel], axis=0)
    out_coords = jnp.concatenate([voxel_coords, voxel_coords], axis=0)
    return (out_features, out_coords)


# speedup vs baseline: 58.0935x; 1.0003x over previous
import functools

import jax
import jax.numpy as jnp
import numpy as np
from jax import lax
from jax.experimental import pallas as pl
from jax.experimental.pallas import tpu as pltpu
from jax.experimental.pallas import tpu_sc as plsc

BS = 2
NC = 6
NPB = 50000
C_IMG = 256
D_OUT = 128
H_F = 16
W_F = 44
DS = 16
N_PTS = BS * NPB
PIX = H_F * W_F
LOC_ZERO = NC * PIX
DSLICE = 16
NDG = D_OUT // DSLICE
NPG = 4
PG_PTS = 25600
N_PAD = NPG * PG_PTS
PROW = 12500
PROWP = 12800
CH = 1024
NCHUNK = PG_PTS // CH
GROUPS = CH // 16
VOXEL_SIZE = np.array([0.1, 0.1, 0.2], dtype=np.float32)
PC_RANGE = np.array([-51.2, -51.2, -5.0], dtype=np.float32)
D_MIN, D_MAX = 1.0, 60.0


def _table_body(f_ref, w_ref, o_ref):
    wb = w_ref[...].astype(jnp.bfloat16)
    for c in range(NC):
        o_ref[0, 0, :, pl.ds(c * PIX, PIX)] = lax.dot_general(
            wb, f_ref[0, c].astype(jnp.bfloat16),
            dimension_numbers=(((1,), (0,)), ((), ())),
            preferred_element_type=jnp.float32,
        )


def _build_tables(feats4d, w):
    return pl.pallas_call(
        _table_body,
        grid=(BS, NDG),
        in_specs=[
            pl.BlockSpec((1, NC, C_IMG, PIX), lambda g, h: (g, 0, 0, 0)),
            pl.BlockSpec((DSLICE, C_IMG), lambda g, h: (h, 0)),
        ],
        out_specs=pl.BlockSpec(
            (1, 1, DSLICE, NC * PIX), lambda g, h: (g, h, 0, 0)),
        out_shape=jax.ShapeDtypeStruct((BS, NDG, DSLICE, NC * PIX),
                                       jnp.float32),
    )(feats4d, w)


def _bf(x):
    return x.astype(jnp.bfloat16).astype(jnp.float32)


def _proj_body(x_ref, y_ref, z_ref, bt_ref, ir_ref, ab_ref, tt_ref, pr_ref,
               pt_ref, out_ref):
    def col(ref, *idx):
        return ref[idx[0], :, pl.ds(idx[1], 1)] if len(idx) == 2 else \
            ref[:, pl.ds(idx[0], 1)]

    x = x_ref[...]
    y = y_ref[...]
    z = z_ref[...]
    x1 = (x * float(VOXEL_SIZE[0]) + float(PC_RANGE[0])) - col(bt_ref, 0)
    y1 = (y * float(VOXEL_SIZE[1]) + float(PC_RANGE[1])) - col(bt_ref, 1)
    z1 = (z * float(VOXEL_SIZE[2]) + float(PC_RANGE[2])) - col(bt_ref, 2)
    xb, yb, zb = _bf(x1), _bf(y1), _bf(z1)
    s0 = xb * col(ir_ref, 0) + yb * col(ir_ref, 1) + zb * col(ir_ref, 2)
    s1 = xb * col(ir_ref, 3) + yb * col(ir_ref, 4) + zb * col(ir_ref, 5)
    s2 = xb * col(ir_ref, 6) + yb * col(ir_ref, 7) + zb * col(ir_ref, 8)
    sb0, sb1, sb2 = _bf(s0), _bf(s1), _bf(s2)
    for c in range(NC):
        p0 = col(tt_ref, c, 0) + sb0 * col(ab_ref, c, 0) + sb1 * col(ab_ref, c, 1) + sb2 * col(ab_ref, c, 2)
        p1 = col(tt_ref, c, 1) + sb0 * col(ab_ref, c, 3) + sb1 * col(ab_ref, c, 4) + sb2 * col(ab_ref, c, 5)
        p2 = col(tt_ref, c, 2) + sb0 * col(ab_ref, c, 6) + sb1 * col(ab_ref, c, 7) + sb2 * col(ab_ref, c, 8)
        u = p0 / p2
        v = p1 / p2
        ub, vb, db = _bf(u), _bf(v), _bf(p2)
        q0 = col(pt_ref, c, 0) + ub * col(pr_ref, c, 0) + vb * col(pr_ref, c, 1) + db * col(pr_ref, c, 2)
        q1 = col(pt_ref, c, 1) + ub * col(pr_ref, c, 3) + vb * col(pr_ref, c, 4) + db * col(pr_ref, c, 5)
        q2 = col(pt_ref, c, 2) + ub * col(pr_ref, c, 6) + vb * col(pr_ref, c, 7) + db * col(pr_ref, c, 8)
        cx = jnp.round(q0 / float(DS))
        cy = jnp.round(q1 / float(DS))
        kept = ((cx >= 0) & (cx < W_F) & (cy >= 0) & (cy < H_F)
                & (q2 < D_MAX) & (q2 >= D_MIN))
        cxi = jnp.clip(jnp.where(jnp.isnan(cx), 0.0, cx), 0.0, W_F - 1.0).astype(jnp.int32)
        cyi = jnp.clip(jnp.where(jnp.isnan(cy), 0.0, cy), 0.0, H_F - 1.0).astype(jnp.int32)
        g = c * PIX + cyi * W_F + cxi
        out_ref[c, :, pl.ds(0, PROW)] = jnp.where(kept, g, LOC_ZERO)
        out_ref[c, :, pl.ds(PROW, PROWP - PROW)] = jnp.full(
            (8, PROWP - PROW), LOC_ZERO, jnp.int32)


def _project_indices(xs, ys, zs, bt, ir, ab, tt, pr, pt):
    return pl.pallas_call(
        _proj_body,
        in_specs=[pl.BlockSpec(memory_space=pltpu.VMEM)] * 9,
        out_specs=pl.BlockSpec(memory_space=pltpu.VMEM),
        out_shape=jax.ShapeDtypeStruct((NC, 8, PROWP), jnp.int32),
    )(xs, ys, zs, bt, ir, ab, tt, pr, pt)


@functools.cache
def _make_sc_gather_acc():
    return functools.partial(
        pl.kernel,
        out_type=jax.ShapeDtypeStruct((D_OUT, N_PAD), jnp.float32),
        mesh=plsc.VectorSubcoreMesh(core_axis_name="c", subcore_axis_name="s"),
        scratch_types=[
            pltpu.VMEM((DSLICE, LOC_ZERO + 16), jnp.float32),
            pltpu.VMEM((2, NC, CH), jnp.int32),
            pltpu.VMEM((2, DSLICE, CH), jnp.float32),
            pltpu.SemaphoreType.DMA,
            pltpu.SemaphoreType.DMA,
        ],
        compiler_params=pltpu.CompilerParams(needs_layout_passes=False),
    )(_sc_body)


def _sc_body(table_hbm, gidx_hbm, out_hbm, tbl_v, idx_v, outs_v, sem_i, sem_o):
    wid = lax.axis_index("s") * 2 + lax.axis_index("c")
    dg = wid % NDG
    pg = wid // NDG
    b = wid // (NDG * 2)
    pltpu.sync_copy(table_hbm.at[b, dg], tbl_v.at[:, pl.ds(0, NC * PIX)])
    zeros16 = jnp.zeros((16,), jnp.float32)
    for r in range(DSLICE):
        tbl_v[r, pl.ds(LOC_ZERO, 16)] = zeros16
    pt0 = pg * PG_PTS

    def fetch(k, buf):
        return pltpu.async_copy(
            gidx_hbm.at[:, pl.ds(pt0 + k * CH, CH)], idx_v.at[buf], sem_i)

    def make_group_body(buf):
        def group_body(g, carry2):
            gsl = pl.ds(g * 16, 16)
            accs = [None] * DSLICE
            for c in range(NC):
                rows = idx_v[buf, c, gsl]
                for d in range(DSLICE):
                    v = plsc.load_gather(
                        tbl_v, [jnp.full((16,), d, jnp.int32), rows])
                    accs[d] = v if c == 0 else accs[d] + v
            for d in range(DSLICE):
                outs_v[buf, d, gsl] = accs[d]
            return carry2
        return group_body

    write_handles = [None] * NCHUNK
    fetch_handles = [None] * NCHUNK
    fetch_handles[0] = fetch(0, 0)
    for k in range(NCHUNK):
        buf = k % 2
        fetch_handles[k].wait()
        if k + 1 < NCHUNK:
            fetch_handles[k + 1] = fetch(k + 1, 1 - buf)
        if k >= 2:
            write_handles[k - 2].wait()
        lax.fori_loop(0, GROUPS, make_group_body(buf), 0)
        write_handles[k] = pltpu.async_copy(
            outs_v.at[buf],
            out_hbm.at[pl.ds(dg * DSLICE, DSLICE), pl.ds(pt0 + k * CH, CH)],
            sem_o)
    write_handles[NCHUNK - 2].wait()
    write_handles[NCHUNK - 1].wait()


def kernel(voxel_features, voxel_coords, img_feats, rots, trans, intrins,
           post_rots, post_trans, bda, lidar2cam, W, imgs):
    f32 = jnp.float32
    bf16 = jnp.bfloat16
    eye4 = jnp.eye(4, dtype=f32)
    c2i = jnp.tile(eye4, (BS, NC, 1, 1))
    c2i = c2i.at[:, :, :3, :3].set(intrins)
    l2i = jnp.einsum("bcij,bckj->bcik", c2i, lidar2cam)
    ab = l2i[:, :, :3, :3].astype(bf16).astype(f32)
    tt = l2i[:, :, :3, 3]
    ir = jnp.linalg.inv(bda[:, :3, :3]).astype(bf16).astype(f32)
    bt = bda[:, :3, 3]
    prb = post_rots.astype(f32).astype(bf16).astype(f32)
    ptf = post_trans.astype(f32)
    bt8 = jnp.repeat(bt, 4, axis=0)
    ir8 = jnp.repeat(ir.reshape(BS, 9), 4, axis=0)
    ab8 = jnp.repeat(ab.transpose(1, 0, 2, 3).reshape(NC, BS, 9), 4, axis=1)
    tt8 = jnp.repeat(tt.transpose(1, 0, 2), 4, axis=1)
    pr8 = jnp.repeat(prb.transpose(1, 0, 2, 3).reshape(NC, BS, 9), 4, axis=1)
    pt8 = jnp.repeat(ptf.transpose(1, 0, 2), 4, axis=1)

    xs = voxel_coords[:, 3].astype(f32).reshape(8, PROW)
    ys = voxel_coords[:, 2].astype(f32).reshape(8, PROW)
    zs = voxel_coords[:, 1].astype(f32).reshape(8, PROW)

    table = _build_tables(img_feats.reshape(BS, NC, C_IMG, PIX), W)

    gidx_pad = _project_indices(xs, ys, zs, bt8, ir8, ab8, tt8, pr8, pt8)

    img_t = _make_sc_gather_acc()(table, gidx_pad.reshape(NC, N_PAD))
    img_voxel = (img_t.reshape(D_OUT, 8, PROWP)[:, :, :PROW]
                 .reshape(D_OUT, N_PTS).T)

    out_features = jnp.concatenate([voxel_features, img_voxel], axis=0)
    out_coords = jnp.concatenate([voxel_coords, voxel_coords], axis=0)
    return (out_features, out_coords)
